# baseline (speedup 1.0000x reference)
import jax
import jax.numpy as jnp
import numpy as np
from jax import lax
from jax.experimental import pallas as pl

_B, _T, _C, _D = 4, 4096, 32, 768
_MAX_LEN = 5000
_NF = 7
_OFFSETS = (0, 13, 45, 52, 54, 78, 80)
_NROWS = 128
_TB = 1024


def _fused_body(x_ref, xt_ref, wt_ref, tab_ref, b_ref, out_ref):
    x_blk = x_ref[...]
    idx = xt_ref[...]
    col = lax.broadcasted_iota(jnp.int32, (_TB, _NROWS), 1)
    oh = jnp.zeros((_TB, _NROWS), jnp.float32)
    for i, off in enumerate(_OFFSETS):
        oh += (col == idx[:, i][:, None] + off).astype(jnp.float32)
    acc = jnp.dot(x_blk, wt_ref[...], preferred_element_type=jnp.float32)
    acc += jnp.dot(oh, tab_ref[...], preferred_element_type=jnp.float32)
    d = lax.broadcasted_iota(jnp.int32, (1, _D), 1)
    half = (d // 2).astype(jnp.float32)
    freq = jnp.exp(half * (-2.0 * np.log(10000.0) / _D))
    phase = (d % 2).astype(jnp.float32) * np.float32(np.pi / 2)
    t0 = pl.program_id(0) % (_T // _TB)
    t = (t0 * _TB + lax.broadcasted_iota(jnp.int32, (_TB, 1), 0)).astype(
        jnp.float32)
    pe = jnp.sin(t * freq + phase)
    out_ref[...] = acc + pe + b_ref[...]


def kernel(x, x_time, W, b, month_tab, day_tab, weekday_tab, holiday_tab,
           hour_tab, event_tab, rain_tab):
    n_tok = _B * _T
    xf = x.reshape(n_tok, _C)
    xt = x_time.reshape(n_tok, _NF)
    wt = W.T
    tab = jnp.concatenate(
        [month_tab, day_tab, weekday_tab, holiday_tab, hour_tab,
         event_tab, rain_tab], axis=0)
    tab = jnp.pad(tab, ((0, _NROWS - tab.shape[0]), (0, 0)))
    n_blk = n_tok // _TB

    out = pl.pallas_call(
        _fused_body,
        grid=(n_blk,),
        in_specs=[
            pl.BlockSpec((_TB, _C), lambda i: (i, 0)),
            pl.BlockSpec((_TB, _NF), lambda i: (i, 0)),
            pl.BlockSpec((_C, _D), lambda i: (0, 0)),
            pl.BlockSpec((_NROWS, _D), lambda i: (0, 0)),
            pl.BlockSpec((1, _D), lambda i: (0, 0)),
        ],
        out_specs=pl.BlockSpec((_TB, _D), lambda i: (i, 0)),
        out_shape=jax.ShapeDtypeStruct((n_tok, _D), jnp.float32),
    )(xf, xt, wt, tab, b.reshape(1, _D))
    return out.reshape(_B, _T, _D)


# speedup vs baseline: 2.4121x; 2.4121x over previous
import jax
import jax.numpy as jnp
import numpy as np
from jax import lax
from jax.experimental import pallas as pl

_B, _T, _C, _D = 4, 4096, 32, 768
_NF = 7
_OFFSETS = (0, 13, 45, 52, 54, 78, 80)
_NROWS = 128
_TB = 1024
_Q = 64


def _pe_factors():
    d = np.arange(_D)
    f = np.exp((d // 2) * (-2.0 * np.log(10000.0) / _D))
    p = (d % 2) * (np.pi / 2)
    q = np.arange(_T // _Q, dtype=np.float64)[:, None]
    r = np.arange(_Q, dtype=np.float64)[:, None]
    ah = _Q * q * f[None, :]
    al = r * f[None, :] + p[None, :]
    mk = lambda a: jnp.asarray(a, dtype=jnp.float32)
    return (mk(np.sin(ah)), mk(np.cos(ah)), mk(np.sin(al)), mk(np.cos(al)))


def _fused_body(x_ref, xt_ref, wt_ref, tab_ref, b_ref,
                sh_ref, ch_ref, sl_ref, cl_ref, out_ref):
    x_blk = x_ref[...]
    idx = xt_ref[...]
    col = lax.broadcasted_iota(jnp.int32, (_TB, _NROWS), 1)
    oh = jnp.zeros((_TB, _NROWS), jnp.float32)
    for i, off in enumerate(_OFFSETS):
        oh += (col == idx[:, i][:, None] + off).astype(jnp.float32)
    acc = jnp.dot(x_blk, wt_ref[...], preferred_element_type=jnp.float32)
    acc += jnp.dot(oh, tab_ref[...], preferred_element_type=jnp.float32)
    t0 = pl.program_id(0) % (_T // _TB) * _TB
    row = lax.broadcasted_iota(jnp.int32, (_TB, 1), 0) + t0
    colq = lax.broadcasted_iota(jnp.int32, (_TB, _T // _Q), 1)
    colr = lax.broadcasted_iota(jnp.int32, (_TB, _Q), 1)
    ohq = (colq == row // _Q).astype(jnp.float32)
    ohr = (colr == row % _Q).astype(jnp.float32)
    sh = jnp.dot(ohq, sh_ref[...], preferred_element_type=jnp.float32)
    ch = jnp.dot(ohq, ch_ref[...], preferred_element_type=jnp.float32)
    sl = jnp.dot(ohr, sl_ref[...], preferred_element_type=jnp.float32)
    cl = jnp.dot(ohr, cl_ref[...], preferred_element_type=jnp.float32)
    out_ref[...] = acc + sh * cl + ch * sl + b_ref[...]


def kernel(x, x_time, W, b, month_tab, day_tab, weekday_tab, holiday_tab,
           hour_tab, event_tab, rain_tab):
    n_tok = _B * _T
    xf = x.reshape(n_tok, _C)
    xt = x_time.reshape(n_tok, _NF)
    wt = W.T
    tab = jnp.concatenate(
        [month_tab, day_tab, weekday_tab, holiday_tab, hour_tab,
         event_tab, rain_tab], axis=0)
    tab = jnp.pad(tab, ((0, _NROWS - tab.shape[0]), (0, 0)))
    sh, ch, sl, cl = _pe_factors()
    n_blk = n_tok // _TB
    nq = _T // _Q

    full = lambda i: (0, 0)
    out = pl.pallas_call(
        _fused_body,
        grid=(n_blk,),
        in_specs=[
            pl.BlockSpec((_TB, _C), lambda i: (i, 0)),
            pl.BlockSpec((_TB, _NF), lambda i: (i, 0)),
            pl.BlockSpec((_C, _D), full),
            pl.BlockSpec((_NROWS, _D), full),
            pl.BlockSpec((1, _D), full),
            pl.BlockSpec((nq, _D), full),
            pl.BlockSpec((nq, _D), full),
            pl.BlockSpec((_Q, _D), full),
            pl.BlockSpec((_Q, _D), full),
        ],
        out_specs=pl.BlockSpec((_TB, _D), lambda i: (i, 0)),
        out_shape=jax.ShapeDtypeStruct((n_tok, _D), jnp.float32),
    )(xf, xt, wt, tab, b.reshape(1, _D), sh, ch, sl, cl)
    return out.reshape(_B, _T, _D)
